# baseline (speedup 1.0000x reference)
import jax
import jax.numpy as jnp
from jax.experimental import pallas as pl
from jax.experimental.pallas import tpu as pltpu


def _encode_kernel(x_ref, w_ref, b_ref, pos_ref, o_ref):
    x = x_ref[0]
    acc = jax.lax.dot_general(
        x, w_ref[...], (((1,), (1,)), ((), ())),
        preferred_element_type=jnp.float32,
    )
    o_ref[0] = acc + b_ref[...] + pos_ref[...]


def kernel(patch, W, b, pos_table):
    B, N, D = patch.shape
    P = W.shape[0]
    b2 = b.reshape(1, P)
    return pl.pallas_call(
        _encode_kernel,
        grid=(B,),
        in_specs=[
            pl.BlockSpec((1, N, D), lambda i: (i, 0, 0)),
            pl.BlockSpec((P, D), lambda i: (0, 0)),
            pl.BlockSpec((1, P), lambda i: (0, 0)),
            pl.BlockSpec((N, P), lambda i: (0, 0)),
        ],
        out_specs=pl.BlockSpec((1, N, P), lambda i: (i, 0, 0)),
        out_shape=jax.ShapeDtypeStruct((B, N, P), jnp.float32),
        compiler_params=pltpu.CompilerParams(
            dimension_semantics=("parallel",),
        ),
    )(patch, W, b2, pos_table)


# speedup vs baseline: 1.0141x; 1.0141x over previous
import jax
import jax.numpy as jnp
from jax.experimental import pallas as pl
from jax.experimental.pallas import tpu as pltpu


def _encode_kernel(x_ref, w_ref, b_ref, pos_ref, o_ref):
    x = x_ref[0].astype(jnp.bfloat16)
    w = w_ref[...].astype(jnp.bfloat16)
    acc = jax.lax.dot_general(
        x, w, (((1,), (1,)), ((), ())),
        preferred_element_type=jnp.float32,
    )
    o_ref[0] = acc + b_ref[...] + pos_ref[...]


def kernel(patch, W, b, pos_table):
    B, N, D = patch.shape
    P = W.shape[0]
    b2 = b.reshape(1, P)
    return pl.pallas_call(
        _encode_kernel,
        grid=(B,),
        in_specs=[
            pl.BlockSpec((1, N, D), lambda i: (i, 0, 0)),
            pl.BlockSpec((P, D), lambda i: (0, 0)),
            pl.BlockSpec((1, P), lambda i: (0, 0)),
            pl.BlockSpec((N, P), lambda i: (0, 0)),
        ],
        out_specs=pl.BlockSpec((1, N, P), lambda i: (i, 0, 0)),
        out_shape=jax.ShapeDtypeStruct((B, N, P), jnp.float32),
        compiler_params=pltpu.CompilerParams(
            dimension_semantics=("parallel",),
        ),
    )(patch, W, b2, pos_table)
